# baseline (speedup 1.0000x reference)
import functools

import jax
import jax.numpy as jnp
from jax import lax
from jax.experimental import pallas as pl
from jax.experimental.pallas import tpu as pltpu
from jax.experimental.pallas import tpu_sc as plsc

N, C, D = 16384, 20, 64
B = N * C
NW = 32
BPW = B // NW
CS = 256
NCHUNK = BPW // CS

_mesh = plsc.VectorSubcoreMesh(core_axis_name="c", subcore_axis_name="s")


def _make_call(v2, spmem_table):
    scratch = [
        pltpu.VMEM((BPW,), jnp.int32),
        [pltpu.VMEM((CS, 128), jnp.float32) for _ in range(2)],
        [pltpu.SemaphoreType.DMA for _ in range(2)],
        [pltpu.SemaphoreType.DMA for _ in range(2)],
    ]
    if spmem_table:
        scratch.append(pltpu.VMEM_SHARED((v2, 128), jnp.float32))

    @functools.partial(
        pl.kernel,
        mesh=_mesh,
        out_type=jax.ShapeDtypeStruct((B, 128), jnp.float32),
        scratch_types=scratch,
        compiler_params=pltpu.CompilerParams(use_tc_tiling_on_sc=False),
    )
    def _call(idx_hbm, w_hbm, out_hbm, idx_v, rows, gsem, wsem, *rest):
        wid = lax.axis_index("s") * 2 + lax.axis_index("c")
        base = wid * BPW

        if spmem_table:
            table = rest[0]

            @pl.when(lax.axis_index("s") == 0)
            def _stage():
                pltpu.sync_copy(w_hbm, table)
        else:
            table = w_hbm

        pltpu.sync_copy(idx_hbm.at[pl.ds(base, BPW)], idx_v)
        if spmem_table:
            plsc.subcore_barrier()

        def gather_desc(g, p):
            return pltpu.make_async_copy(
                table.at[idx_v.at[pl.ds(g * CS, CS)]], rows[p], gsem[p])

        def write_desc(g, p):
            return pltpu.make_async_copy(
                rows[p], out_hbm.at[pl.ds(base + g * CS, CS)], wsem[p])

        gather_desc(0, 0).start()
        gather_desc(1, 1).start()
        gather_desc(0, 0).wait()
        write_desc(0, 0).start()

        def body(t, carry):
            g0 = 2 * t
            write_desc(g0 - 2, 0).wait()
            gather_desc(g0, 0).start()
            gather_desc(g0 - 1, 1).wait()
            write_desc(g0 - 1, 1).start()
            write_desc(g0 - 1, 1).wait()
            gather_desc(g0 + 1, 1).start()
            gather_desc(g0, 0).wait()
            write_desc(g0, 0).start()
            return carry

        lax.fori_loop(1, NCHUNK // 2, body, 0)

        gather_desc(NCHUNK - 1, 1).wait()
        write_desc(NCHUNK - 1, 1).start()
        write_desc(NCHUNK - 2, 0).wait()
        write_desc(NCHUNK - 1, 1).wait()

    return _call


_call_a = _make_call(500000, False)
_call_b = _make_call(50000, False)
_call_c = _make_call(500, True)


def _select(wide, cat):
    p = (cat.reshape(B) & 1)[:, None] == 1
    sel = jnp.where(p, wide[:, 64:128], wide[:, 0:64])
    return sel.reshape(N, C, D)


def kernel(cat_a, cat_b, cat_c, W_cat_a, W_cat_b, W_cat_c):
    ia = (cat_a.astype(jnp.int32) >> 1).reshape(B)
    ib = (cat_b.astype(jnp.int32) >> 1).reshape(B)
    ic = (cat_c.astype(jnp.int32) >> 1).reshape(B)
    wa = W_cat_a.reshape(500000, 128)
    wb = W_cat_b.reshape(50000, 128)
    wc = W_cat_c.reshape(500, 128)
    ow_a = _call_a(ia, wa)
    ow_b = _call_b(ib, wb)
    ow_c = _call_c(ic, wc)
    return (_select(ow_a, cat_a), _select(ow_b, cat_b), _select(ow_c, cat_c))


# speedup vs baseline: 1.6442x; 1.6442x over previous
import functools

import jax
import jax.numpy as jnp
from jax import lax
from jax.experimental import pallas as pl
from jax.experimental.pallas import tpu as pltpu
from jax.experimental.pallas import tpu_sc as plsc

N, C, D = 16384, 20, 64
B = N * C
NW = 32
BPW = B // NW
CS = 256
NCHUNK = BPW // CS

_mesh = plsc.VectorSubcoreMesh(core_axis_name="c", subcore_axis_name="s")


def _make_call(vocab, spmem_table):
    scratch = [
        pltpu.VMEM((BPW,), jnp.int32),
        [pltpu.VMEM((CS, D), jnp.float32) for _ in range(2)],
        [pltpu.SemaphoreType.DMA for _ in range(2)],
        [pltpu.SemaphoreType.DMA for _ in range(2)],
    ]
    if spmem_table:
        scratch.append(pltpu.VMEM_SHARED((vocab, D), jnp.float32))

    @functools.partial(
        pl.kernel,
        mesh=_mesh,
        out_type=jax.ShapeDtypeStruct((B, D), jnp.float32),
        scratch_types=scratch,
        compiler_params=pltpu.CompilerParams(use_tc_tiling_on_sc=False),
    )
    def _call(idx_hbm, w_hbm, out_hbm, idx_v, rows, gsem, wsem, *rest):
        wid = lax.axis_index("s") * 2 + lax.axis_index("c")
        base = wid * BPW

        if spmem_table:
            table = rest[0]

            @pl.when(lax.axis_index("s") == 0)
            def _stage():
                pltpu.sync_copy(w_hbm, table)
        else:
            table = w_hbm

        pltpu.sync_copy(idx_hbm.at[pl.ds(base, BPW)], idx_v)
        if spmem_table:
            plsc.subcore_barrier()

        def gather_desc(g, p):
            return pltpu.make_async_copy(
                table.at[idx_v.at[pl.ds(g * CS, CS)]], rows[p], gsem[p])

        def write_desc(g, p):
            return pltpu.make_async_copy(
                rows[p], out_hbm.at[pl.ds(base + g * CS, CS)], wsem[p])

        gather_desc(0, 0).start()
        gather_desc(1, 1).start()
        gather_desc(0, 0).wait()
        write_desc(0, 0).start()

        def body(t, carry):
            g0 = 2 * t
            write_desc(g0 - 2, 0).wait()
            gather_desc(g0, 0).start()
            gather_desc(g0 - 1, 1).wait()
            write_desc(g0 - 1, 1).start()
            write_desc(g0 - 1, 1).wait()
            gather_desc(g0 + 1, 1).start()
            gather_desc(g0, 0).wait()
            write_desc(g0, 0).start()
            return carry

        lax.fori_loop(1, NCHUNK // 2, body, 0)

        gather_desc(NCHUNK - 1, 1).wait()
        write_desc(NCHUNK - 1, 1).start()
        write_desc(NCHUNK - 2, 0).wait()
        write_desc(NCHUNK - 1, 1).wait()

    return _call


_call_a = _make_call(1000000, False)
_call_b = _make_call(100000, False)
_call_c = _make_call(1000, True)


def kernel(cat_a, cat_b, cat_c, W_cat_a, W_cat_b, W_cat_c):
    ia = cat_a.reshape(B).astype(jnp.int32)
    ib = cat_b.reshape(B).astype(jnp.int32)
    ic = cat_c.reshape(B).astype(jnp.int32)
    oa = _call_a(ia, W_cat_a)
    ob = _call_b(ib, W_cat_b)
    oc = _call_c(ic, W_cat_c)
    return (oa.reshape(N, C, D), ob.reshape(N, C, D), oc.reshape(N, C, D))
